# baseline (speedup 1.0000x reference)
import functools
import math

import jax
import jax.numpy as jnp
from jax import lax
from jax.experimental import pallas as pl
from jax.experimental.pallas import tpu as pltpu
from jax.experimental.pallas import tpu_sc as plsc

_NC = 2
_NS = 16
_NW = _NC * _NS
_L = 16

_LN2 = math.log(2.0)
_X_MAX = 100.0
_ALPHA = 0.75
_SQRT2 = math.sqrt(2.0)

_V = 100000
_B = 16384
_D = 64
_NBLK = (_V + 127) // 128
_VFULL = (_V // 128) * 128
_KMAX = (_NBLK + _NW - 1) // _NW
_DUMP = 2 * _B
_ROWS = 2 * _B + 8


def _ln(x):
    y = lax.bitcast_convert_type(x, jnp.int32)
    e = lax.shift_right_logical(y, 23) - 127
    m = lax.bitcast_convert_type(
        (y & jnp.int32(0x007FFFFF)) | jnp.int32(0x3F800000), jnp.float32)
    big = m > _SQRT2
    m = jnp.where(big, 0.5 * m, m)
    ef = e.astype(jnp.float32) + jnp.where(big, 1.0, 0.0)
    s = (m - 1.0) / (m + 1.0)
    t = s * s
    ln_m = 2.0 * s * (1.0 + t * (1.0 / 3.0 + t * (0.2 + t * (1.0 / 7.0 + t / 9.0))))
    return ef * _LN2 + ln_m


def _sc_gather(c_idx, p_idx, cT, pT, tail_c, tail_p, rowbuf,
               tch_v, hv_v, hp_v, bcol_v, bpos_v, blk_v, tcl_v,
               hstage_v, pos2_v, sem):
    wid = lax.axis_index("s") * _NC + lax.axis_index("c")
    lane = lax.iota(jnp.int32, _L)

    for tab, tail, idx_hbm, off in ((cT, tail_c, c_idx, 0),
                                    (pT, tail_p, p_idx, _B)):
        def comp_outer(ch, cnt):
            pltpu.sync_copy(idx_hbm.at[pl.ds(ch * 512, 512)], tch_v)

            def comp_inner(j, cnt):
                v = tch_v[pl.ds(j * _L, _L)]
                pos = ch * 512 + j * _L + lane + off
                m = lax.shift_right_logical(v, 7) % _NW == wid
                plsc.store_compressed(hv_v.at[pl.ds(cnt, _L)], v, mask=m)
                plsc.store_compressed(hp_v.at[pl.ds(cnt, _L)], pos, mask=m)
                return cnt + plsc.all_reduce_population_count(m)[0]

            return lax.fori_loop(0, 512 // _L, comp_inner, cnt)

        cnt = lax.fori_loop(0, _B // 512, comp_outer, jnp.int32(0))
        nch = (cnt + _L - 1) // _L

        def serve_block(k, fill):
            blk = wid + k * _NW
            base = blk * 128

            @pl.when(blk < _NBLK - 1)
            def _():
                pltpu.sync_copy(tab.at[:, pl.ds(base, 128)],
                                blk_v.at[:, pl.ds(0, 128)])

            @pl.when(blk == _NBLK - 1)
            def _():
                pltpu.sync_copy(tail, tcl_v)

                def tail_tr(cc, x):
                    flat = cc * _L + lane
                    plsc.store_scatter(
                        blk_v, [flat & (_D - 1), lax.shift_right_logical(flat, 6)],
                        tcl_v[pl.ds(cc * _L, _L)])
                    return x

                lax.fori_loop(0, (_V - _VFULL) * _D // _L, tail_tr, 0)

            def rescan(c2, bfill):
                hv = hv_v[pl.ds(c2 * _L, _L)]
                hp = hp_v[pl.ds(c2 * _L, _L)]
                mb = (lax.shift_right_logical(hv, 7) == blk) & (
                    c2 * _L + lane < cnt)
                plsc.store_compressed(bcol_v.at[pl.ds(bfill, _L)], hv & 127, mask=mb)
                plsc.store_compressed(bpos_v.at[pl.ds(bfill, _L)], hp, mask=mb)
                return bfill + plsc.all_reduce_population_count(mb)[0]

            bfill = lax.fori_loop(0, nch, rescan, jnp.int32(0))
            bcol_v[pl.ds(bfill, _L)] = jnp.zeros((_L,), jnp.int32)
            bpos_v[pl.ds(bfill, _L)] = jnp.full((_L,), _DUMP, jnp.int32)
            nserve = (bfill + _L - 1) // _L

            def srv(s2, fill):
                cols = bcol_v[pl.ds(s2 * _L, _L)]
                poss = bpos_v[pl.ds(s2 * _L, _L)]
                for i in range(_L):
                    csp = jnp.take_along_axis(
                        cols, jnp.full((_L,), i, jnp.int32), axis=0)
                    for j in range(_D // _L):
                        g = plsc.load_gather(blk_v, [lane + j * _L, csp])
                        hstage_v[fill + i, pl.ds(j * _L, _L)] = g
                pos2_v[0, pl.ds(fill, _L)] = poss
                fill = fill + _L

                @pl.when(fill == 128)
                def _():
                    pltpu.async_copy(
                        hstage_v, rowbuf.at[pos2_v.at[0]], sem).wait()

                return jnp.where(fill == 128, 0, fill)

            return lax.fori_loop(0, nserve, srv, fill)

        fill = lax.fori_loop(0, _KMAX, serve_block, jnp.int32(0))

        def padpos(c3, x):
            @pl.when(c3 * _L >= fill)
            def _():
                pos2_v[0, pl.ds(c3 * _L, _L)] = jnp.full((_L,), _DUMP,
                                                         jnp.int32)
            return x

        lax.fori_loop(0, 128 // _L, padpos, 0)

        @pl.when(fill > 0)
        def _():
            pltpu.async_copy(hstage_v, rowbuf.at[pos2_v.at[0]], sem).wait()


def _sc_loss(rowbuf, c_idx, p_idx, labels, c_bias, p_bias, out,
             cidx_v, pidx_v, lab_v, ce_v, pe_v, cb_v, pb_v, stage_v, tr_v,
             sem_ce, sem_pe, sem_cb, sem_pb):
    per = lab_v.shape[0]
    nch = per // 128
    wid = lax.axis_index("s") * _NC + lax.axis_index("c")
    base = wid * per

    for k in range(nch):
        pltpu.sync_copy(c_idx.at[pl.ds(base + k * 128, 128)], cidx_v.at[k])
        pltpu.sync_copy(p_idx.at[pl.ds(base + k * 128, 128)], pidx_v.at[k])
    pltpu.sync_copy(labels.at[pl.ds(base, per)], lab_v)

    handles = []
    for k in range(nch):
        rows = pl.ds(k * 128, 128)
        handles.append(pltpu.async_copy(
            c_bias.at[cidx_v.at[k]], cb_v.at[rows], sem_cb))
        handles.append(pltpu.async_copy(
            p_bias.at[pidx_v.at[k]], pb_v.at[rows], sem_pb))
    for h in handles:
        h.wait()

    lane = lax.iota(jnp.int32, _L)
    lane17 = lane * 17
    acc = jnp.zeros((_L,), jnp.float32)

    for c in range(nch):
        pltpu.async_copy(
            rowbuf.at[pl.ds(base + c * 128, 128)], ce_v, sem_ce).wait()
        pltpu.async_copy(
            rowbuf.at[pl.ds(_B + base + c * 128, 128)], pe_v, sem_pe).wait()

        def body(g, acc, c=c):
            gbase = g * _L
            for j in range(_L):
                r = gbase + j
                prod = ce_v[r, pl.ds(0, _L)] * pe_v[r, pl.ds(0, _L)]
                for k in range(1, _D // _L):
                    prod = prod + ce_v[r, pl.ds(k * _L, _L)] * pe_v[r, pl.ds(k * _L, _L)]
                plsc.store_scatter(tr_v, [lane17 + j], prod)
            dots = tr_v[pl.ds(0, _L)]
            for i in range(1, _L):
                dots = dots + tr_v[pl.ds(i * 17, _L)]
            gl = c * 128 + gbase
            l = lab_v[pl.ds(gl, _L)]
            cb = cb_v[pl.ds(gl, _L)]
            pb = pb_v[pl.ds(gl, _L)]
            lnl = _ln(l)
            w = jnp.minimum(jnp.exp(_ALPHA * (lnl - math.log(_X_MAX))), 1.0)
            diff = dots + cb + pb - lnl
            return acc + w * diff * diff

        acc = lax.fori_loop(0, 128 // _L, body, acc)

    stage_v[...] = acc
    pltpu.sync_copy(stage_v, out.at[wid])


def _tc_mean(p_ref, o_ref, *, inv_n):
    o_ref[...] = jnp.sum(p_ref[...], keepdims=True) * inv_n


def kernel(c_data, p_data, labels, c_embed, c_bias, p_embed, p_bias):
    per = _B // _NW

    ci = c_data.astype(jnp.int32)
    pi = p_data.astype(jnp.int32)
    cb1 = c_bias.reshape(_V)
    pb1 = p_bias.reshape(_V)

    mesh = plsc.VectorSubcoreMesh(core_axis_name="c", subcore_axis_name="s")
    params = pltpu.CompilerParams(needs_layout_passes=False)

    gather = functools.partial(
        pl.kernel,
        mesh=mesh,
        out_type=jax.ShapeDtypeStruct((_ROWS, 128), jnp.float32),
        scratch_types=[
            pltpu.VMEM((512,), jnp.int32),
            pltpu.VMEM((_B + _L,), jnp.int32),
            pltpu.VMEM((_B + _L,), jnp.int32),
            pltpu.VMEM((_B + 2 * _L,), jnp.int32),
            pltpu.VMEM((_B + 2 * _L,), jnp.int32),
            pltpu.VMEM((_D, 129), jnp.float32),
            pltpu.VMEM(((_V - _VFULL) * _D,), jnp.float32),
            pltpu.VMEM((128, 128), jnp.float32),
            pltpu.VMEM((1, 128), jnp.int32),
            pltpu.SemaphoreType.DMA,
        ],
        compiler_params=params,
    )(_sc_gather)
    tail_c = c_embed[_VFULL:, :].reshape(-1)
    tail_p = p_embed[_VFULL:, :].reshape(-1)
    rowbuf = gather(ci, pi, c_embed.T, p_embed.T, tail_c, tail_p)

    loss_k = functools.partial(
        pl.kernel,
        mesh=mesh,
        out_type=jax.ShapeDtypeStruct((_NW, _L), jnp.float32),
        scratch_types=[
            pltpu.VMEM((per // 128, 128), jnp.int32),
            pltpu.VMEM((per // 128, 128), jnp.int32),
            pltpu.VMEM((per,), jnp.float32),
            pltpu.VMEM((128, 128), jnp.float32),
            pltpu.VMEM((128, 128), jnp.float32),
            pltpu.VMEM((per,), jnp.float32),
            pltpu.VMEM((per,), jnp.float32),
            pltpu.VMEM((_L,), jnp.float32),
            pltpu.VMEM((_L * 17,), jnp.float32),
            pltpu.SemaphoreType.DMA,
            pltpu.SemaphoreType.DMA,
            pltpu.SemaphoreType.DMA,
            pltpu.SemaphoreType.DMA,
        ],
        compiler_params=params,
    )(_sc_loss)
    parts = loss_k(rowbuf, ci, pi, labels, cb1, pb1)

    loss = pl.pallas_call(
        functools.partial(_tc_mean, inv_n=1.0 / _B),
        out_shape=jax.ShapeDtypeStruct((1, 1), jnp.float32),
    )(parts)
    return loss[0, 0]


# speedup vs baseline: 2.4981x; 2.4981x over previous
import functools
import math

import jax
import jax.numpy as jnp
from jax import lax
from jax.experimental import pallas as pl
from jax.experimental.pallas import tpu as pltpu
from jax.experimental.pallas import tpu_sc as plsc

_NC = 2
_NS = 16
_NW = _NC * _NS
_L = 16

_LN2 = math.log(2.0)
_X_MAX = 100.0
_ALPHA = 0.75
_SQRT2 = math.sqrt(2.0)

_V = 100000
_B = 16384
_D = 64
_NBLK = (_V + 127) // 128
_VFULL = (_V // 128) * 128
_KMAX = (_NBLK + _NW - 1) // _NW
_PR = _V // 2
_TS = 130


def _ln(x):
    y = lax.bitcast_convert_type(x, jnp.int32)
    e = lax.shift_right_logical(y, 23) - 127
    m = lax.bitcast_convert_type(
        (y & jnp.int32(0x007FFFFF)) | jnp.int32(0x3F800000), jnp.float32)
    big = m > _SQRT2
    m = jnp.where(big, 0.5 * m, m)
    ef = e.astype(jnp.float32) + jnp.where(big, 1.0, 0.0)
    s = (m - 1.0) / (m + 1.0)
    t = s * s
    ln_m = 2.0 * s * (1.0 + t * (1.0 / 3.0 + t * (0.2 + t * (1.0 / 7.0 + t / 9.0))))
    return ef * _LN2 + ln_m


def _sc_transpose(cT, pT, tail_c, tail_p, out_c, out_p,
                  blk0_v, blk1_v, tr0_v, tr1_v, tcl_v, sem_in, sem_out):
    wid = lax.axis_index("s") * _NC + lax.axis_index("c")
    lane = lax.iota(jnp.int32, _L)

    rowidx = []
    colbase = []
    for j in range(128 // _L):
        vv = lane + j * _L
        rowidx.append(lax.shift_right_logical(vv, 1))
        colbase.append((vv & 1) * _D)

    for tab, tail, out in ((cT, tail_c, out_c), (pT, tail_p, out_p)):
        blks = (blk0_v, blk1_v)
        trs = (tr0_v, tr1_v)

        def start_in(k, buf):
            blk = wid + k * _NW

            @pl.when(blk < _NBLK - 1)
            def _():
                pltpu.async_copy(tab.at[:, pl.ds(blk * 128, 128)], buf, sem_in)

        def transpose(buf, tr):
            def tr_d(d, x):
                for j in range(128 // _L):
                    plsc.store_scatter(tr, [rowidx[j], colbase[j] + d],
                                       buf[d, pl.ds(j * _L, _L)])
                return x

            lax.fori_loop(0, _D, tr_d, 0)

        def process(k, b):
            blk = wid + k * _NW
            start_in(k + 1, blks[1 - b])

            @pl.when(blk < _NBLK - 1)
            def _():
                pltpu.make_async_copy(
                    tab.at[:, pl.ds(blk * 128, 128)], blks[b], sem_in
                ).wait()
                transpose(blks[b], trs[b])
                pltpu.async_copy(trs[b].at[:, pl.ds(0, 128)],
                                 out.at[pl.ds(blk * 64, 64)], sem_out).wait()

        start_in(0, blks[0])

        def round2(k2, x):
            process(k2 * 2, 0)
            process(k2 * 2 + 1, 1)
            return x

        lax.fori_loop(0, (_KMAX + 1) // 2, round2, 0)

        @pl.when(wid == (_NBLK - 1) % _NW)
        def _():
            pltpu.sync_copy(tail, tcl_v)

            def tail_tr(cc, x):
                flat = cc * _L + lane
                v = lax.shift_right_logical(flat, 6)
                d = flat & (_D - 1)
                plsc.store_scatter(
                    tr0_v, [lax.shift_right_logical(v, 1), (v & 1) * _D + d],
                    tcl_v[pl.ds(cc * _L, _L)])
                return x

            lax.fori_loop(0, (_V - _VFULL) * _D // _L, tail_tr, 0)
            pltpu.async_copy(tr0_v.at[pl.ds(0, 16), pl.ds(0, 128)],
                             out.at[pl.ds(_VFULL // 2, 16)], sem_out).wait()


def _sc_loss(prc, prp, c_idx, p_idx, labels, c_bias, p_bias, out,
             cidx_v, pidx_v, cpr_v, ppr_v, lab_v, ce_v, pe_v, cb_v, pb_v,
             stage_v, tr_v, sem_ce, sem_pe, sem_cb, sem_pb):
    per = lab_v.shape[0]
    nch = per // 128
    wid = lax.axis_index("s") * _NC + lax.axis_index("c")
    base = wid * per

    for k in range(nch):
        pltpu.sync_copy(c_idx.at[pl.ds(base + k * 128, 128)], cidx_v.at[k])
        pltpu.sync_copy(p_idx.at[pl.ds(base + k * 128, 128)], pidx_v.at[k])
    pltpu.sync_copy(labels.at[pl.ds(base, per)], lab_v)

    def mk_pr(j, x):
        for k in range(nch):
            v = cidx_v[k, pl.ds(j * _L, _L)]
            cpr_v[k, pl.ds(j * _L, _L)] = lax.shift_right_logical(v, 1)
            w = pidx_v[k, pl.ds(j * _L, _L)]
            ppr_v[k, pl.ds(j * _L, _L)] = lax.shift_right_logical(w, 1)
        return x

    lax.fori_loop(0, 128 // _L, mk_pr, 0)

    handles = []
    for k in range(nch):
        rows = pl.ds(k * 128, 128)
        handles.append(pltpu.async_copy(
            c_bias.at[cidx_v.at[k]], cb_v.at[rows], sem_cb))
        handles.append(pltpu.async_copy(
            p_bias.at[pidx_v.at[k]], pb_v.at[rows], sem_pb))
    for h in handles:
        h.wait()

    lane = lax.iota(jnp.int32, _L)
    lane17 = lane * 17
    acc = jnp.zeros((_L,), jnp.float32)

    hc = pltpu.async_copy(prc.at[cpr_v.at[0]], ce_v.at[0], sem_ce)
    hp = pltpu.async_copy(prp.at[ppr_v.at[0]], pe_v.at[0], sem_pe)

    for c in range(nch):
        hc.wait()
        hp.wait()
        if c + 1 < nch:
            hc = pltpu.async_copy(
                prc.at[cpr_v.at[c + 1]], ce_v.at[(c + 1) % 2], sem_ce)
            hp = pltpu.async_copy(
                prp.at[ppr_v.at[c + 1]], pe_v.at[(c + 1) % 2], sem_pe)
        cebuf = ce_v.at[c % 2]
        pebuf = pe_v.at[c % 2]

        def body(g, acc, c=c, cebuf=cebuf, pebuf=pebuf):
            gbase = g * _L
            cparv = (cidx_v[c, pl.ds(gbase, _L)] & 1) * _D
            pparv = (pidx_v[c, pl.ds(gbase, _L)] & 1) * _D
            for j in range(_L):
                r = gbase + j
                co = jnp.take_along_axis(cparv, jnp.full((_L,), j, jnp.int32),
                                         axis=0) + lane
                po = jnp.take_along_axis(pparv, jnp.full((_L,), j, jnp.int32),
                                         axis=0) + lane
                prod = None
                for k in range(_D // _L):
                    cv = plsc.load_gather(cebuf, [jnp.full((_L,), r, jnp.int32),
                                                  co + k * _L])
                    pv = plsc.load_gather(pebuf, [jnp.full((_L,), r, jnp.int32),
                                                  po + k * _L])
                    prod = cv * pv if prod is None else prod + cv * pv
                plsc.store_scatter(tr_v, [lane17 + j], prod)
            dots = tr_v[pl.ds(0, _L)]
            for i in range(1, _L):
                dots = dots + tr_v[pl.ds(i * 17, _L)]
            gl = c * 128 + gbase
            l = lab_v[pl.ds(gl, _L)]
            cb = cb_v[pl.ds(gl, _L)]
            pb = pb_v[pl.ds(gl, _L)]
            lnl = _ln(l)
            w = jnp.minimum(jnp.exp(_ALPHA * (lnl - math.log(_X_MAX))), 1.0)
            diff = dots + cb + pb - lnl
            return acc + w * diff * diff

        acc = lax.fori_loop(0, 128 // _L, body, acc)

    stage_v[...] = acc
    pltpu.sync_copy(stage_v, out.at[wid])


def _tc_mean(p_ref, o_ref, *, inv_n):
    o_ref[...] = jnp.sum(p_ref[...], keepdims=True) * inv_n


def kernel(c_data, p_data, labels, c_embed, c_bias, p_embed, p_bias):
    per = _B // _NW

    ci = c_data.astype(jnp.int32)
    pi = p_data.astype(jnp.int32)
    cb1 = c_bias.reshape(_V)
    pb1 = p_bias.reshape(_V)

    mesh = plsc.VectorSubcoreMesh(core_axis_name="c", subcore_axis_name="s")
    params = pltpu.CompilerParams(needs_layout_passes=False)

    tr_k = functools.partial(
        pl.kernel,
        mesh=mesh,
        out_type=(jax.ShapeDtypeStruct((_PR, 128), jnp.float32),
                  jax.ShapeDtypeStruct((_PR, 128), jnp.float32)),
        scratch_types=[
            pltpu.VMEM((_D, 128), jnp.float32),
            pltpu.VMEM((_D, 128), jnp.float32),
            pltpu.VMEM((_D, _TS), jnp.float32),
            pltpu.VMEM((_D, _TS), jnp.float32),
            pltpu.VMEM(((_V - _VFULL) * _D,), jnp.float32),
            pltpu.SemaphoreType.DMA,
            pltpu.SemaphoreType.DMA,
        ],
        compiler_params=params,
    )(_sc_transpose)
    tail_c = c_embed[_VFULL:, :].reshape(-1)
    tail_p = p_embed[_VFULL:, :].reshape(-1)
    prc, prp = tr_k(c_embed.T, p_embed.T, tail_c, tail_p)

    loss_k = functools.partial(
        pl.kernel,
        mesh=mesh,
        out_type=jax.ShapeDtypeStruct((_NW, _L), jnp.float32),
        scratch_types=[
            pltpu.VMEM((per // 128, 128), jnp.int32),
            pltpu.VMEM((per // 128, 128), jnp.int32),
            pltpu.VMEM((per // 128, 128), jnp.int32),
            pltpu.VMEM((per // 128, 128), jnp.int32),
            pltpu.VMEM((per,), jnp.float32),
            pltpu.VMEM((2, 128, 128), jnp.float32),
            pltpu.VMEM((2, 128, 128), jnp.float32),
            pltpu.VMEM((per,), jnp.float32),
            pltpu.VMEM((per,), jnp.float32),
            pltpu.VMEM((_L,), jnp.float32),
            pltpu.VMEM((_L * 17,), jnp.float32),
            pltpu.SemaphoreType.DMA,
            pltpu.SemaphoreType.DMA,
            pltpu.SemaphoreType.DMA,
            pltpu.SemaphoreType.DMA,
        ],
        compiler_params=params,
    )(_sc_loss)
    parts = loss_k(prc, prp, ci, pi, labels, cb1, pb1)

    loss = pl.pallas_call(
        functools.partial(_tc_mean, inv_n=1.0 / _B),
        out_shape=jax.ShapeDtypeStruct((1, 1), jnp.float32),
    )(parts)
    return loss[0, 0]


# speedup vs baseline: 5.2430x; 2.0988x over previous
import functools
import math

import jax
import jax.numpy as jnp
from jax import lax
from jax.experimental import pallas as pl
from jax.experimental.pallas import tpu as pltpu
from jax.experimental.pallas import tpu_sc as plsc

_NC = 2
_NS = 16
_NW = _NC * _NS
_L = 16

_LN2 = math.log(2.0)
_X_MAX = 100.0
_ALPHA = 0.75
_SQRT2 = math.sqrt(2.0)


def _ln(x):
    y = lax.bitcast_convert_type(x, jnp.int32)
    e = lax.shift_right_logical(y, 23) - 127
    m = lax.bitcast_convert_type(
        (y & jnp.int32(0x007FFFFF)) | jnp.int32(0x3F800000), jnp.float32)
    big = m > _SQRT2
    m = jnp.where(big, 0.5 * m, m)
    ef = e.astype(jnp.float32) + jnp.where(big, 1.0, 0.0)
    s = (m - 1.0) / (m + 1.0)
    t = s * s
    ln_m = 2.0 * s * (1.0 + t * (1.0 / 3.0 + t * (0.2 + t * (1.0 / 7.0 + t / 9.0))))
    return ef * _LN2 + ln_m


def _sc_glove(c_idx, p_idx, labels, c_embed, c_bias, p_embed, p_bias,
              out, cidx_v, pidx_v, lab_v, ce_v, pe_v, cb_v, pb_v, stage_v,
              tr_v, sem_ce, sem_pe, sem_cb, sem_pb):
    per = lab_v.shape[0]
    nch = cidx_v.shape[0]
    dim = ce_v.shape[1]
    wid = lax.axis_index("s") * _NC + lax.axis_index("c")
    base = wid * per

    for k in range(nch):
        pltpu.sync_copy(c_idx.at[pl.ds(base + k * 128, 128)], cidx_v.at[k])
        pltpu.sync_copy(p_idx.at[pl.ds(base + k * 128, 128)], pidx_v.at[k])
    pltpu.sync_copy(labels.at[pl.ds(base, per)], lab_v)

    handles = []
    for k in range(nch):
        rows = pl.ds(k * 128, 128)
        handles.append(pltpu.async_copy(c_embed.at[cidx_v.at[k]], ce_v.at[rows], sem_ce))
        handles.append(pltpu.async_copy(p_embed.at[pidx_v.at[k]], pe_v.at[rows], sem_pe))
        handles.append(pltpu.async_copy(c_bias.at[cidx_v.at[k]], cb_v.at[rows], sem_cb))
        handles.append(pltpu.async_copy(p_bias.at[pidx_v.at[k]], pb_v.at[rows], sem_pb))
    for h in handles:
        h.wait()

    lane = lax.iota(jnp.int32, _L)
    lane17 = lane * 17
    nd = dim // _L

    def body(g, acc):
        base = g * _L
        for j in range(_L):
            r = base + j
            prod = ce_v[r, pl.ds(0, _L)] * pe_v[r, pl.ds(0, _L)]
            for k in range(1, nd):
                prod = prod + ce_v[r, pl.ds(k * _L, _L)] * pe_v[r, pl.ds(k * _L, _L)]
            plsc.store_scatter(tr_v, [lane17 + j], prod)
        dots = tr_v[pl.ds(0, _L)]
        for i in range(1, _L):
            dots = dots + tr_v[pl.ds(i * 17, _L)]
        l = lab_v[pl.ds(base, _L)]
        cb = cb_v[pl.ds(base, _L)]
        pb = pb_v[pl.ds(base, _L)]
        lnl = _ln(l)
        w = jnp.minimum(jnp.exp(_ALPHA * (lnl - math.log(_X_MAX))), 1.0)
        diff = dots + cb + pb - lnl
        return acc + w * diff * diff

    acc = lax.fori_loop(0, per // _L, body, jnp.zeros((_L,), jnp.float32))
    stage_v[...] = acc
    pltpu.sync_copy(stage_v, out.at[wid])


def _tc_mean(p_ref, o_ref, *, inv_n):
    o_ref[...] = jnp.sum(p_ref[...], keepdims=True) * inv_n


def kernel(c_data, p_data, labels, c_embed, c_bias, p_embed, p_bias):
    batch = c_data.shape[0]
    vocab, dim = c_embed.shape
    per = batch // _NW
    nch = per // 128

    ci = c_data.astype(jnp.int32)
    pi = p_data.astype(jnp.int32)
    cb1 = c_bias.reshape(vocab)
    pb1 = p_bias.reshape(vocab)

    sc = functools.partial(
        pl.kernel,
        mesh=plsc.VectorSubcoreMesh(core_axis_name="c", subcore_axis_name="s"),
        out_type=jax.ShapeDtypeStruct((_NW, _L), jnp.float32),
        compiler_params=pltpu.CompilerParams(
            needs_layout_passes=False, use_tc_tiling_on_sc=False),
        scratch_types=[
            pltpu.VMEM((nch, 128), jnp.int32),
            pltpu.VMEM((nch, 128), jnp.int32),
            pltpu.VMEM((per,), jnp.float32),
            pltpu.VMEM((per, dim), jnp.float32),
            pltpu.VMEM((per, dim), jnp.float32),
            pltpu.VMEM((per,), jnp.float32),
            pltpu.VMEM((per,), jnp.float32),
            pltpu.VMEM((_L,), jnp.float32),
            pltpu.VMEM((_L * 17,), jnp.float32),
            pltpu.SemaphoreType.DMA,
            pltpu.SemaphoreType.DMA,
            pltpu.SemaphoreType.DMA,
            pltpu.SemaphoreType.DMA,
        ],
    )(_sc_glove)
    parts = sc(ci, pi, labels, c_embed, cb1, p_embed, pb1)

    loss = pl.pallas_call(
        functools.partial(_tc_mean, inv_n=1.0 / batch),
        out_shape=jax.ShapeDtypeStruct((1, 1), jnp.float32),
    )(parts)
    return loss[0, 0]
